# baseline (speedup 1.0000x reference)
import functools
import numpy as np
import jax
import jax.numpy as jnp
from jax import lax
from jax.experimental import pallas as pl
from jax.experimental.pallas import tpu as pltpu, tpu_sc as plsc


def _build_idx_list():
    num_candidates = 16
    indices = [0, 1, 2, 3, 4, 5, 6, 7, 8]
    base_idx = 9
    for i in range(num_candidates - 1):
        indices += [6, 7, base_idx + i]
    indices += [0, 3, 6, 1, 4, 7, 2, 5, 8]
    for i in range(num_candidates - 1):
        indices += [2, 5, base_idx + i]
    return indices


_IDX = _build_idx_list()


def _merge_runs(idx):
    runs = []
    o_start, s_start, length = 0, idx[0], 1
    for j in range(1, len(idx)):
        if idx[j] == s_start + length:
            length += 1
        else:
            runs.append((o_start, s_start, length))
            o_start, s_start, length = j, idx[j], 1
    runs.append((o_start, s_start, length))
    return runs


_RUNS = _merge_runs(_IDX)


def _tc_body(x_ref, o_hbm, sem):
    b = pl.program_id(0)
    copies = [
        pltpu.make_async_copy(
            x_ref.at[0, pl.ds(s_start, length)],
            o_hbm.at[b, pl.ds(o_start, length)],
            sem,
        )
        for o_start, s_start, length in _RUNS
    ]
    for c in copies:
        c.start()
    for c in copies:
        c.wait()


_CPS = 8
_ROWW = 8192
_SC_ROWS = 108 * _CPS
_NW = 32
_NSLAB = 108
_GMAX = (_NSLAB + _NW - 1) // _NW


def _sc_src_rows():
    idx = np.array(_IDX, dtype=np.int32)
    c = np.arange(_CPS, dtype=np.int32)
    src = idx[:, None] * _CPS + c[None, :]
    pad = np.zeros((_GMAX * _NW - _NSLAB, _CPS), dtype=np.int32)
    src = np.concatenate([src, pad], axis=0)
    return src.reshape(_GMAX, _NW, _CPS).transpose(1, 0, 2).copy()


_SC_SRC = _sc_src_rows()


def _sc_copy(x_flat, src):
    mesh = plsc.VectorSubcoreMesh(core_axis_name="c", subcore_axis_name="s")

    @functools.partial(
        pl.kernel,
        mesh=mesh,
        out_type=jax.ShapeDtypeStruct((_SC_ROWS, _ROWW), jnp.float32),
        scratch_types=[
            pltpu.VMEM((_GMAX, _CPS), jnp.int32),
            pltpu.VMEM((_CPS, _ROWW), jnp.float32),
            pltpu.SemaphoreType.DMA,
        ],
    )
    def k(x_hbm, src_hbm, out_hbm, idx_v, buf_v, sem):
        wid = lax.axis_index("s") * 2 + lax.axis_index("c")
        pltpu.sync_copy(src_hbm.at[wid], idx_v)

        for g in range(_GMAX):
            slab = g * _NW + wid

            @pl.when(slab < _NSLAB)
            def _():
                pltpu.async_copy(x_hbm.at[idx_v.at[g]], buf_v, sem).wait()
                pltpu.sync_copy(buf_v, out_hbm.at[pl.ds(slab * _CPS, _CPS)])

    return k(x_flat, src)


def kernel(x):
    b, n, s, d = x.shape
    n_out = len(_IDX)

    tc_out = pl.pallas_call(
        _tc_body,
        grid=(b - 1,),
        in_specs=[pl.BlockSpec((1, n, s, d), lambda i: (i, 0, 0, 0))],
        out_specs=pl.BlockSpec(memory_space=pl.ANY),
        out_shape=jax.ShapeDtypeStruct((b - 1, n_out, s, d), x.dtype),
        scratch_shapes=[pltpu.SemaphoreType.DMA],
    )(x[: b - 1])

    sc_out = _sc_copy(x[b - 1].reshape(n * _CPS, _ROWW), jnp.asarray(_SC_SRC))
    sc_out = sc_out.reshape(1, n_out, s, d)

    out = jnp.concatenate([tc_out, sc_out], axis=0)
    return out.reshape(b, n_out // 3, 3, s, d)


# speedup vs baseline: 3.9454x; 3.9454x over previous
import jax
import jax.numpy as jnp
from jax.experimental import pallas as pl
from jax.experimental.pallas import tpu as pltpu


def _build_idx_list():
    num_candidates = 16
    indices = [0, 1, 2, 3, 4, 5, 6, 7, 8]
    base_idx = 9
    for i in range(num_candidates - 1):
        indices += [6, 7, base_idx + i]
    indices += [0, 3, 6, 1, 4, 7, 2, 5, 8]
    for i in range(num_candidates - 1):
        indices += [2, 5, base_idx + i]
    return indices


_IDX = _build_idx_list()


def _merge_runs(idx):
    runs = []
    o_start, s_start, length = 0, idx[0], 1
    for j in range(1, len(idx)):
        if idx[j] == s_start + length:
            length += 1
        else:
            runs.append((o_start, s_start, length))
            o_start, s_start, length = j, idx[j], 1
    runs.append((o_start, s_start, length))
    return runs


_RUNS = _merge_runs(_IDX)


_BPB = 2


def _body(x_ref, o_hbm, sem):
    i = pl.program_id(0)
    copies = [
        pltpu.make_async_copy(
            x_ref.at[bb, pl.ds(s_start, length)],
            o_hbm.at[i * _BPB + bb, pl.ds(o_start, length)],
            sem,
        )
        for bb in range(_BPB)
        for o_start, s_start, length in _RUNS
    ]
    for c in copies:
        c.start()
    for c in copies:
        c.wait()


def kernel(x):
    b, n, s, d = x.shape
    n_out = len(_IDX)

    out = pl.pallas_call(
        _body,
        grid=(b // _BPB,),
        in_specs=[pl.BlockSpec((_BPB, n, s, d), lambda i: (i, 0, 0, 0))],
        out_specs=pl.BlockSpec(memory_space=pl.ANY),
        out_shape=jax.ShapeDtypeStruct((b, n_out, s, d), x.dtype),
        scratch_shapes=[pltpu.SemaphoreType.DMA],
    )(x)
    return out.reshape(b, n_out // 3, 3, s, d)
